# baseline (speedup 1.0000x reference)
import functools
import jax
import jax.numpy as jnp
from jax import lax
from jax.experimental import pallas as pl
from jax.experimental.pallas import tpu as pltpu
from jax.experimental.pallas import tpu_sc as plsc

_TILE = 7168


def _pass1_body(x_ref, w1_ref, b1_ref, w2_ref, b2_ref, out0_ref, fac_ref, imp_ref):
    xb = x_ref[0]
    h1 = lax.dot_general(w1_ref[...].astype(jnp.bfloat16), xb.astype(jnp.bfloat16),
                         (((1,), (0,)), ((), ())),
                         preferred_element_type=jnp.float32)
    h1 = jnp.maximum(h1 + b1_ref[...], 0.0)
    s = lax.dot_general(w2_ref[...].astype(jnp.bfloat16), h1.astype(jnp.bfloat16),
                        (((1,), (0,)), ((), ())),
                        preferred_element_type=jnp.float32) + b2_ref[...]
    sm = jax.nn.sigmoid(s)
    nh = sm.shape[0]
    c = xb.shape[0]
    fac0v = (0.7 / nh) * jnp.sum(sm, axis=0)
    fac_ref[0, 0, :] = fac0v
    imp_ref[0, 0, :] = jnp.sum(jnp.abs(xb), axis=0) * (1.0 / c)
    out0_ref[0] = xb * fac0v[None, :]


def _make_sc_finish(num_sel, hw, nb, nch):
    info = plsc.get_sparse_core_info()
    nc, ns, lanes = info.num_cores, info.num_subcores, info.num_lanes
    assert nb == nc
    chunk = next(c for c in range(128, hw + 1, 128)
                 if hw % c == 0 and hw // c <= ns)
    active = hw // chunk
    nv = chunk // lanes
    kv = nch // lanes
    _UNROLL = next(u for u in (16, 8, 4, 2, 1) if nv % u == 0)
    idx_bits = max(hw - 1, 1).bit_length()
    nbits = 31 + idx_bits
    add = 0.3 / num_sel
    mesh = plsc.VectorSubcoreMesh(core_axis_name="c", subcore_axis_name="s")

    def _lane_total(v):
        total = v[0]
        for l in range(1, lanes):
            total = total + v[l]
        return total

    @functools.partial(
        pl.kernel, mesh=mesh,
        out_type=(),
        scratch_types=[
            pltpu.VMEM((chunk,), jnp.float32),
            pltpu.VMEM((chunk,), jnp.float32),
            pltpu.SMEM((2 * (31 + max(hw - 1, 1).bit_length()),), jnp.int32),
            pltpu.VMEM((nch,), jnp.float32),
            pltpu.SemaphoreType.DMA,
        ])
    def sc_finish(imp_hbm, fac0_hbm, out_ref, impv, facv,
                  cnt_smem, colbuf, sem):
        c = lax.axis_index("c")
        s = lax.axis_index("s")
        is_active = s < active
        base = jnp.where(is_active, s, 0) * chunk

        @pl.when(is_active)
        def _load():
            pltpu.sync_copy(imp_hbm.at[c, 0, pl.ds(base, chunk)], impv)
            pltpu.sync_copy(fac0_hbm.at[c, 0, pl.ds(base, chunk)], facv)

        @pl.when(s == 0)
        def _zero():
            def zbody(k, carry):
                cnt_smem[k] = 0
                return carry
            lax.fori_loop(0, 2 * nbits, zbody, 0)
        plsc.subcore_barrier()

        lane_iota = lax.broadcasted_iota(jnp.uint32, (lanes,), 0)
        inv_base = (jnp.uint32(hw - 1) - jnp.full((lanes,), base, jnp.uint32)
                    - lane_iota)

        def take_mask(j, tv, ti):
            v = impv[pl.ds(j * lanes, lanes)]
            u = lax.bitcast_convert_type(v, jnp.uint32)
            inv = inv_base - jnp.uint32(lanes) * jnp.uint32(j)
            return (u > tv) | ((u == tv) & (inv >= ti))

        def bit_body(i, carry):
            tv, ti = carry
            bit = jnp.int32(nbits - 1) - i
            sh_v = jnp.maximum(bit - idx_bits, 0).astype(jnp.uint32)
            sh_i = jnp.maximum(bit, 0).astype(jnp.uint32)
            mv = jnp.where(bit >= idx_bits, jnp.uint32(1) << sh_v, jnp.uint32(0))
            mi = jnp.where(bit < idx_bits, jnp.uint32(1) << sh_i, jnp.uint32(0))
            try_v = tv | mv
            try_i = ti | mi

            def cbody(jj, acc):
                for u in range(_UNROLL):
                    tk = take_mask(jj * _UNROLL + u, try_v, try_i)
                    acc = acc + jnp.where(tk, 1, 0).astype(jnp.int32)
                return acc

            acc = lax.fori_loop(0, nv // _UNROLL, cbody,
                                jnp.zeros((lanes,), jnp.int32))
            total = acc[0]
            for l in range(1, lanes):
                total = total + acc[l]
            total = jnp.where(is_active, total, 0)
            slot = i + c * nbits
            plsc.fetch_and_add(cnt_smem.at[slot], total, subcore_id=0)
            plsc.subcore_barrier()
            g = plsc.fetch_and_add(cnt_smem.at[slot], 0, subcore_id=0)
            keepm = jnp.full((lanes,), g, jnp.int32) >= num_sel
            return jnp.where(keepm, try_v, tv), jnp.where(keepm, try_i, ti)

        z = jnp.zeros((lanes,), jnp.uint32)
        tv, ti = lax.fori_loop(0, nbits, bit_body, (z, z))

        col_off = c * (nch * hw)
        ch_iota = lax.broadcasted_iota(jnp.int32, (lanes,), 0)

        def fixvreg(j, carry):
            tk = take_mask(j, tv, ti)
            tki = jnp.where(tk, 1, 0).astype(jnp.int32)

            @pl.when(_lane_total(tki) > 0)
            def _():
                f0 = facv[pl.ds(j * lanes, lanes)]
                ratio = (f0 + jnp.float32(add)) / f0
                for l in range(lanes):
                    @pl.when(tki[l] > 0)
                    def _fix(l=l):
                        p = base + j * lanes + l
                        rat = jnp.full((lanes,), ratio[l], jnp.float32)
                        p_spl = jnp.full((lanes,), p, jnp.int32)
                        gets = []
                        for k in range(kv):
                            idxv = (col_off + (ch_iota + k * lanes) * hw
                                    + p_spl)
                            gets.append(pltpu.async_copy(
                                out_ref.at[idxv],
                                colbuf.at[pl.ds(k * lanes, lanes)], sem))
                        for g in gets:
                            g.wait()
                        for k in range(kv):
                            colbuf[pl.ds(k * lanes, lanes)] = (
                                colbuf[pl.ds(k * lanes, lanes)] * rat)
                        puts = []
                        for k in range(kv):
                            idxv = (col_off + (ch_iota + k * lanes) * hw
                                    + p_spl)
                            puts.append(pltpu.async_copy(
                                colbuf.at[pl.ds(k * lanes, lanes)],
                                out_ref.at[idxv], sem))
                        for g in puts:
                            g.wait()
            return carry

        @pl.when(is_active)
        def _fixup():
            lax.fori_loop(0, nv, fixvreg, 0)

    return sc_finish


def kernel(x, gm_w1, gm_b1, gm_w2, gm_b2, qkv_w, ge_w1, ge_b1, ge_w2, ge_b2):
    del qkv_w, ge_w1, ge_b1, ge_w2, ge_b2
    bb, c, h, w = x.shape
    hw = h * w
    hid = gm_w1.shape[0]
    nh = gm_w2.shape[0]
    num_sel = min(max(1, int(hw * 0.01)), 64)
    tile = _TILE if hw % _TILE == 0 else hw

    xf = x.reshape(bb, c, hw)
    b1 = gm_b1.reshape(hid, 1)
    b2 = gm_b2.reshape(nh, 1)

    out0, fac0, imp = pl.pallas_call(
        _pass1_body,
        grid=(bb, hw // tile),
        in_specs=[
            pl.BlockSpec((1, c, tile), lambda b, t: (b, 0, t)),
            pl.BlockSpec((hid, c), lambda b, t: (0, 0)),
            pl.BlockSpec((hid, 1), lambda b, t: (0, 0)),
            pl.BlockSpec((nh, hid), lambda b, t: (0, 0)),
            pl.BlockSpec((nh, 1), lambda b, t: (0, 0)),
        ],
        out_specs=[
            pl.BlockSpec((1, c, tile), lambda b, t: (b, 0, t)),
            pl.BlockSpec((1, 1, tile), lambda b, t: (b, 0, t)),
            pl.BlockSpec((1, 1, tile), lambda b, t: (b, 0, t)),
        ],
        out_shape=[
            jax.ShapeDtypeStruct((bb, c, hw), jnp.float32),
            jax.ShapeDtypeStruct((bb, 1, hw), jnp.float32),
            jax.ShapeDtypeStruct((bb, 1, hw), jnp.float32),
        ],
    )(xf, gm_w1, b1, gm_w2, b2)

    out_ref = jax.new_ref(out0.reshape(bb * c * hw))
    _make_sc_finish(num_sel, hw, bb, c)(imp, fac0, out_ref)
    return out_ref[...].reshape(bb, c, h, w)


# speedup vs baseline: 1.5341x; 1.5341x over previous
import functools
import jax
import jax.numpy as jnp
from jax import lax
from jax.experimental import pallas as pl
from jax.experimental.pallas import tpu as pltpu
from jax.experimental.pallas import tpu_sc as plsc

_TILE = 7168


def _pass1_body(x_ref, w1_ref, b1_ref, w2_ref, b2_ref, out0_ref, fac_ref, imp_ref):
    xb = x_ref[0]
    h1 = lax.dot_general(w1_ref[...].astype(jnp.bfloat16), xb.astype(jnp.bfloat16),
                         (((1,), (0,)), ((), ())),
                         preferred_element_type=jnp.float32)
    h1 = jnp.maximum(h1 + b1_ref[...], 0.0)
    s = lax.dot_general(w2_ref[...].astype(jnp.bfloat16), h1.astype(jnp.bfloat16),
                        (((1,), (0,)), ((), ())),
                        preferred_element_type=jnp.float32) + b2_ref[...]
    sm = jax.nn.sigmoid(s)
    nh = sm.shape[0]
    c = xb.shape[0]
    fac0v = (0.7 / nh) * jnp.sum(sm, axis=0)
    fac_ref[0, 0, :] = fac0v
    imp_ref[0, 0, :] = jnp.sum(jnp.abs(xb), axis=0) * (1.0 / c)
    out0_ref[0] = xb * fac0v[None, :]


def _make_sc_finish(num_sel, hw, nb, nch):
    info = plsc.get_sparse_core_info()
    nc, ns, lanes = info.num_cores, info.num_subcores, info.num_lanes
    assert nb == nc
    chunk = next(c for c in range(128, hw + 1, 128)
                 if hw % c == 0 and hw // c <= ns)
    active = hw // chunk
    nv = chunk // lanes
    kv = nch // lanes
    _UNROLL = next(u for u in (16, 8, 4, 2, 1) if nv % u == 0)
    idx_bits = max(hw - 1, 1).bit_length()
    nbits = 31 + idx_bits
    add = 0.3 / num_sel
    mesh = plsc.VectorSubcoreMesh(core_axis_name="c", subcore_axis_name="s")

    def _lane_total(v):
        total = v[0]
        for l in range(1, lanes):
            total = total + v[l]
        return total

    @functools.partial(
        pl.kernel, mesh=mesh,
        out_type=(),
        scratch_types=[
            pltpu.VMEM((chunk,), jnp.float32),
            pltpu.VMEM((chunk,), jnp.float32),
            pltpu.SMEM((2 * (31 + max(hw - 1, 1).bit_length()),), jnp.int32),
            pltpu.VMEM((nch // lanes, lanes, 128), jnp.float32),
            pltpu.VMEM((128,), jnp.int32),
            pltpu.VMEM((128,), jnp.float32),
            pltpu.SemaphoreType.DMA,
        ])
    def sc_finish(imp_hbm, fac0_hbm, out_ref, impv, facv,
                  cnt_smem, colbuf, selbuf, ratbuf, sem):
        c = lax.axis_index("c")
        s = lax.axis_index("s")
        is_active = s < active
        base = jnp.where(is_active, s, 0) * chunk

        @pl.when(is_active)
        def _load():
            pltpu.sync_copy(imp_hbm.at[c, 0, pl.ds(base, chunk)], impv)
            pltpu.sync_copy(fac0_hbm.at[c, 0, pl.ds(base, chunk)], facv)

        @pl.when(s == 0)
        def _zero():
            def zbody(k, carry):
                cnt_smem[k] = 0
                return carry
            lax.fori_loop(0, 2 * nbits, zbody, 0)
        plsc.subcore_barrier()

        lane_iota = lax.broadcasted_iota(jnp.uint32, (lanes,), 0)
        inv_base = (jnp.uint32(hw - 1) - jnp.full((lanes,), base, jnp.uint32)
                    - lane_iota)

        def take_mask(j, tv, ti):
            v = impv[pl.ds(j * lanes, lanes)]
            u = lax.bitcast_convert_type(v, jnp.uint32)
            inv = inv_base - jnp.uint32(lanes) * jnp.uint32(j)
            return (u > tv) | ((u == tv) & (inv >= ti))

        def bit_body(i, carry):
            tv, ti = carry
            bit = jnp.int32(nbits - 1) - i
            sh_v = jnp.maximum(bit - idx_bits, 0).astype(jnp.uint32)
            sh_i = jnp.maximum(bit, 0).astype(jnp.uint32)
            mv = jnp.where(bit >= idx_bits, jnp.uint32(1) << sh_v, jnp.uint32(0))
            mi = jnp.where(bit < idx_bits, jnp.uint32(1) << sh_i, jnp.uint32(0))
            try_v = tv | mv
            try_i = ti | mi

            def cbody(jj, acc):
                for u in range(_UNROLL):
                    tk = take_mask(jj * _UNROLL + u, try_v, try_i)
                    acc = acc + jnp.where(tk, 1, 0).astype(jnp.int32)
                return acc

            acc = lax.fori_loop(0, nv // _UNROLL, cbody,
                                jnp.zeros((lanes,), jnp.int32))
            total = acc[0]
            for l in range(1, lanes):
                total = total + acc[l]
            total = jnp.where(is_active, total, 0)
            slot = i + c * nbits
            plsc.fetch_and_add(cnt_smem.at[slot], total, subcore_id=0)
            plsc.subcore_barrier()
            g = plsc.fetch_and_add(cnt_smem.at[slot], 0, subcore_id=0)
            keepm = jnp.full((lanes,), g, jnp.int32) >= num_sel
            return jnp.where(keepm, try_v, tv), jnp.where(keepm, try_i, ti)

        z = jnp.zeros((lanes,), jnp.uint32)
        tv, ti = lax.fori_loop(0, nbits, bit_body, (z, z))

        lane_i32 = lax.broadcasted_iota(jnp.int32, (lanes,), 0)

        def _lane_scan(j, tki, ptr):
            f0 = facv[pl.ds(j * lanes, lanes)]
            ratio = (f0 + jnp.float32(add)) / f0
            for l in range(lanes):
                selbuf[pl.ds(ptr, lanes)] = jnp.full(
                    (lanes,), base + j * lanes + l, jnp.int32)
                ratbuf[pl.ds(ptr, lanes)] = jnp.full(
                    (lanes,), ratio[l], jnp.float32)
                ptr = ptr + tki[l]
            return ptr

        def cmpbody(j, ptr):
            tk = take_mask(j, tv, ti)
            tki = jnp.where(tk, 1, 0).astype(jnp.int32)
            return lax.cond(_lane_total(tki) > 0,
                            lambda q: _lane_scan(j, tki, q),
                            lambda q: q, ptr)

        n_local = lax.cond(
            is_active,
            lambda: lax.fori_loop(0, nv, cmpbody, jnp.int32(0)),
            lambda: jnp.int32(0))

        def fixone(i, carry):
            @pl.when(i < n_local)
            def _():
                p = selbuf[pl.ds(i, lanes)][0]
                rat = ratbuf[pl.ds(i, lanes)][0]
                w0 = pl.multiple_of((p // 128) * 128, 128)
                colq = p - w0
                gbase = (colq // lanes) * lanes
                lq = colq - gbase
                rvec = jnp.where(lane_i32 == lq, rat, jnp.float32(1.0))
                gets = []
                for k in range(kv):
                    gets.append(pltpu.async_copy(
                        out_ref.at[c, pl.ds(k * lanes, lanes),
                                   pl.ds(w0, 128)],
                        colbuf.at[k], sem))
                for g in gets:
                    g.wait()
                for k in range(kv):
                    for r in range(lanes):
                        colbuf[k, r, pl.ds(gbase, lanes)] = (
                            colbuf[k, r, pl.ds(gbase, lanes)] * rvec)
                puts = []
                for k in range(kv):
                    puts.append(pltpu.async_copy(
                        colbuf.at[k],
                        out_ref.at[c, pl.ds(k * lanes, lanes),
                                   pl.ds(w0, 128)], sem))
                for g in puts:
                    g.wait()
            return carry

        lax.fori_loop(0, num_sel, fixone, 0)

    return sc_finish


def kernel(x, gm_w1, gm_b1, gm_w2, gm_b2, qkv_w, ge_w1, ge_b1, ge_w2, ge_b2):
    del qkv_w, ge_w1, ge_b1, ge_w2, ge_b2
    bb, c, h, w = x.shape
    hw = h * w
    hid = gm_w1.shape[0]
    nh = gm_w2.shape[0]
    num_sel = min(max(1, int(hw * 0.01)), 64)
    tile = _TILE if hw % _TILE == 0 else hw

    xf = x.reshape(bb, c, hw)
    b1 = gm_b1.reshape(hid, 1)
    b2 = gm_b2.reshape(nh, 1)

    out0, fac0, imp = pl.pallas_call(
        _pass1_body,
        grid=(bb, hw // tile),
        in_specs=[
            pl.BlockSpec((1, c, tile), lambda b, t: (b, 0, t)),
            pl.BlockSpec((hid, c), lambda b, t: (0, 0)),
            pl.BlockSpec((hid, 1), lambda b, t: (0, 0)),
            pl.BlockSpec((nh, hid), lambda b, t: (0, 0)),
            pl.BlockSpec((nh, 1), lambda b, t: (0, 0)),
        ],
        out_specs=[
            pl.BlockSpec((1, c, tile), lambda b, t: (b, 0, t)),
            pl.BlockSpec((1, 1, tile), lambda b, t: (b, 0, t)),
            pl.BlockSpec((1, 1, tile), lambda b, t: (b, 0, t)),
        ],
        out_shape=[
            jax.ShapeDtypeStruct((bb, c, hw), jnp.float32),
            jax.ShapeDtypeStruct((bb, 1, hw), jnp.float32),
            jax.ShapeDtypeStruct((bb, 1, hw), jnp.float32),
        ],
    )(xf, gm_w1, b1, gm_w2, b2)

    out_ref = jax.new_ref(out0)
    _make_sc_finish(num_sel, hw, bb, c)(imp, fac0, out_ref)
    return out_ref[...].reshape(bb, c, h, w)


# speedup vs baseline: 1.8295x; 1.1926x over previous
import functools
import jax
import jax.numpy as jnp
from jax import lax
from jax.experimental import pallas as pl
from jax.experimental.pallas import tpu as pltpu
from jax.experimental.pallas import tpu_sc as plsc

_TILE = 7168


def _pass1_body(x_ref, w1_ref, b1_ref, w2_ref, b2_ref, out0_ref, fac_ref, imp_ref):
    xb = x_ref[0]
    h1 = lax.dot_general(w1_ref[...].astype(jnp.bfloat16), xb.astype(jnp.bfloat16),
                         (((1,), (0,)), ((), ())),
                         preferred_element_type=jnp.float32)
    h1 = jnp.maximum(h1 + b1_ref[...], 0.0)
    s = lax.dot_general(w2_ref[...].astype(jnp.bfloat16), h1.astype(jnp.bfloat16),
                        (((1,), (0,)), ((), ())),
                        preferred_element_type=jnp.float32) + b2_ref[...]
    sm = jax.nn.sigmoid(s)
    nh = sm.shape[0]
    c = xb.shape[0]
    fac0v = (0.7 / nh) * jnp.sum(sm, axis=0)
    fac_ref[0, 0, :] = fac0v
    imp_ref[0, 0, :] = jnp.sum(jnp.abs(xb), axis=0) * (1.0 / c)
    out0_ref[0] = xb * fac0v[None, :]


def _make_sc_finish(num_sel, hw, nb, nch):
    info = plsc.get_sparse_core_info()
    nc, ns, lanes = info.num_cores, info.num_subcores, info.num_lanes
    assert nb == nc
    chunk = next(c for c in range(128, hw + 1, 128)
                 if hw % c == 0 and hw // c <= ns)
    active = hw // chunk
    nv = chunk // lanes
    kv = nch // lanes
    _UNROLL = next(u for u in (32, 16, 8, 4, 2, 1) if nv % u == 0)
    idx_bits = max(hw - 1, 1).bit_length()
    nbits = 31 + idx_bits
    add = 0.3 / num_sel
    mesh = plsc.VectorSubcoreMesh(core_axis_name="c", subcore_axis_name="s")

    def _lane_total(v):
        total = v[0]
        for l in range(1, lanes):
            total = total + v[l]
        return total

    @functools.partial(
        pl.kernel, mesh=mesh,
        out_type=(),
        scratch_types=[
            pltpu.VMEM((chunk,), jnp.float32),
            pltpu.VMEM((chunk,), jnp.float32),
            pltpu.SMEM((2 * (31 + max(hw - 1, 1).bit_length()),), jnp.int32),
            pltpu.VMEM((nch // lanes, lanes, 128), jnp.float32),
            pltpu.VMEM((128,), jnp.int32),
            pltpu.VMEM((128,), jnp.float32),
            pltpu.SemaphoreType.DMA,
        ])
    def sc_finish(imp_hbm, fac0_hbm, out_ref, impv, facv,
                  cnt_smem, colbuf, selbuf, ratbuf, sem):
        c = lax.axis_index("c")
        s = lax.axis_index("s")
        is_active = s < active
        base = jnp.where(is_active, s, 0) * chunk

        @pl.when(is_active)
        def _load():
            pltpu.sync_copy(imp_hbm.at[c, 0, pl.ds(base, chunk)], impv)
            pltpu.sync_copy(fac0_hbm.at[c, 0, pl.ds(base, chunk)], facv)

        @pl.when(s == 0)
        def _zero():
            def zbody(k, carry):
                cnt_smem[k] = 0
                return carry
            lax.fori_loop(0, 2 * nbits, zbody, 0)
        plsc.subcore_barrier()

        lane_iota = lax.broadcasted_iota(jnp.uint32, (lanes,), 0)
        inv_base = (jnp.uint32(hw - 1) - jnp.full((lanes,), base, jnp.uint32)
                    - lane_iota)

        def take_mask(j, tv, ti):
            v = impv[pl.ds(j * lanes, lanes)]
            u = lax.bitcast_convert_type(v, jnp.uint32)
            inv = inv_base - jnp.uint32(lanes) * jnp.uint32(j)
            return (u > tv) | ((u == tv) & (inv >= ti))

        def _lane_total(v):
            total = v[0]
            for l in range(1, lanes):
                total = total + v[l]
            return total

        def exchange(slot, total):
            plsc.fetch_and_add(cnt_smem.at[slot], total, subcore_id=0)
            plsc.subcore_barrier()
            return plsc.fetch_and_add(cnt_smem.at[slot], 0, subcore_id=0)

        def vbody(i, carry):
            tv, g = carry
            vb = jnp.int32(30) - i
            try_v = tv | (jnp.uint32(1) << vb.astype(jnp.uint32))

            def cbody(jj, acc):
                for u in range(_UNROLL):
                    vv = impv[pl.ds((jj * _UNROLL + u) * lanes, lanes)]
                    uu = lax.bitcast_convert_type(vv, jnp.uint32)
                    acc = acc + jnp.where(uu >= try_v, 1, 0).astype(jnp.int32)
                return acc

            acc = lax.fori_loop(0, nv // _UNROLL, cbody,
                                jnp.zeros((lanes,), jnp.int32))
            total = jnp.where(is_active, _lane_total(acc), 0)
            gt = exchange(i + c * nbits, total)
            keep = gt >= num_sel
            keepm = jnp.full((lanes,), jnp.where(keep, 1, 0), jnp.int32) > 0
            return jnp.where(keepm, try_v, tv), jnp.where(keep, gt, g)

        z = jnp.zeros((lanes,), jnp.uint32)
        tv, gv = lax.fori_loop(0, 31, vbody, (z, jnp.int32(hw)))

        def ibody(i, ti):
            ib = jnp.int32(idx_bits - 1) - i
            try_i = ti | (jnp.uint32(1) << ib.astype(jnp.uint32))

            def cbody(jj, acc):
                for u in range(_UNROLL):
                    tk = take_mask(jj * _UNROLL + u, tv, try_i)
                    acc = acc + jnp.where(tk, 1, 0).astype(jnp.int32)
                return acc

            acc = lax.fori_loop(0, nv // _UNROLL, cbody,
                                jnp.zeros((lanes,), jnp.int32))
            total = jnp.where(is_active, _lane_total(acc), 0)
            gt = exchange(31 + i + c * nbits, total)
            keepm = jnp.full((lanes,), gt, jnp.int32) >= num_sel
            return jnp.where(keepm, try_i, ti)

        n_idx = jnp.where(gv == num_sel, 0, idx_bits)
        ti = lax.fori_loop(0, n_idx, ibody, z)

        lane_i32 = lax.broadcasted_iota(jnp.int32, (lanes,), 0)

        def _lane_scan(j, tki, ptr):
            f0 = facv[pl.ds(j * lanes, lanes)]
            ratio = (f0 + jnp.float32(add)) / f0
            for l in range(lanes):
                selbuf[pl.ds(ptr, lanes)] = jnp.full(
                    (lanes,), base + j * lanes + l, jnp.int32)
                ratbuf[pl.ds(ptr, lanes)] = jnp.full(
                    (lanes,), ratio[l], jnp.float32)
                ptr = ptr + tki[l]
            return ptr

        def cmpbody(j, ptr):
            tk = take_mask(j, tv, ti)
            tki = jnp.where(tk, 1, 0).astype(jnp.int32)
            return lax.cond(_lane_total(tki) > 0,
                            lambda q: _lane_scan(j, tki, q),
                            lambda q: q, ptr)

        n_local = lax.cond(
            is_active,
            lambda: lax.fori_loop(0, nv, cmpbody, jnp.int32(0)),
            lambda: jnp.int32(0))

        def fixone(i, carry):
            @pl.when(i < n_local)
            def _():
                p = selbuf[pl.ds(i, lanes)][0]
                rat = ratbuf[pl.ds(i, lanes)][0]
                w0 = pl.multiple_of((p // 128) * 128, 128)
                colq = p - w0
                gbase = (colq // lanes) * lanes
                lq = colq - gbase
                rvec = jnp.where(lane_i32 == lq, rat, jnp.float32(1.0))
                gets = []
                for k in range(kv):
                    gets.append(pltpu.async_copy(
                        out_ref.at[c, pl.ds(k * lanes, lanes),
                                   pl.ds(w0, 128)],
                        colbuf.at[k], sem))
                for g in gets:
                    g.wait()
                for k in range(kv):
                    for r in range(lanes):
                        colbuf[k, r, pl.ds(gbase, lanes)] = (
                            colbuf[k, r, pl.ds(gbase, lanes)] * rvec)
                puts = []
                for k in range(kv):
                    puts.append(pltpu.async_copy(
                        colbuf.at[k],
                        out_ref.at[c, pl.ds(k * lanes, lanes),
                                   pl.ds(w0, 128)], sem))
                for g in puts:
                    g.wait()
            return carry

        lax.fori_loop(0, num_sel, fixone, 0)

    return sc_finish


def kernel(x, gm_w1, gm_b1, gm_w2, gm_b2, qkv_w, ge_w1, ge_b1, ge_w2, ge_b2):
    del qkv_w, ge_w1, ge_b1, ge_w2, ge_b2
    bb, c, h, w = x.shape
    hw = h * w
    hid = gm_w1.shape[0]
    nh = gm_w2.shape[0]
    num_sel = min(max(1, int(hw * 0.01)), 64)
    tile = _TILE if hw % _TILE == 0 else hw

    xf = x.reshape(bb, c, hw)
    b1 = gm_b1.reshape(hid, 1)
    b2 = gm_b2.reshape(nh, 1)

    out0, fac0, imp = pl.pallas_call(
        _pass1_body,
        grid=(bb, hw // tile),
        in_specs=[
            pl.BlockSpec((1, c, tile), lambda b, t: (b, 0, t)),
            pl.BlockSpec((hid, c), lambda b, t: (0, 0)),
            pl.BlockSpec((hid, 1), lambda b, t: (0, 0)),
            pl.BlockSpec((nh, hid), lambda b, t: (0, 0)),
            pl.BlockSpec((nh, 1), lambda b, t: (0, 0)),
        ],
        out_specs=[
            pl.BlockSpec((1, c, tile), lambda b, t: (b, 0, t)),
            pl.BlockSpec((1, 1, tile), lambda b, t: (b, 0, t)),
            pl.BlockSpec((1, 1, tile), lambda b, t: (b, 0, t)),
        ],
        out_shape=[
            jax.ShapeDtypeStruct((bb, c, hw), jnp.float32),
            jax.ShapeDtypeStruct((bb, 1, hw), jnp.float32),
            jax.ShapeDtypeStruct((bb, 1, hw), jnp.float32),
        ],
    )(xf, gm_w1, b1, gm_w2, b2)

    out_ref = jax.new_ref(out0)
    _make_sc_finish(num_sel, hw, bb, c)(imp, fac0, out_ref)
    return out_ref[...].reshape(bb, c, h, w)
